# baseline (speedup 1.0000x reference)
import functools

import jax
import jax.numpy as jnp
from jax import lax
from jax.experimental import pallas as pl
from jax.experimental.pallas import tpu as pltpu, tpu_sc as plsc

N = 16777216
HALF = N // 2
_INFO = plsc.get_sparse_core_info()
NC = _INFO.num_cores
NS = _INFO.num_subcores
L = _INFO.num_lanes
NW = NC * NS
PER_W = HALF // NW
CHUNK = 8192
NCHUNK = PER_W // CHUNK
DEPTH = 4

_mesh = plsc.VectorSubcoreMesh(core_axis_name="c", subcore_axis_name="s")


@functools.partial(
    pl.kernel,
    out_type=jax.ShapeDtypeStruct((HALF,), jnp.float32),
    mesh=_mesh,
    scratch_types=[
        [pltpu.VMEM((CHUNK,), jnp.float32) for _ in range(DEPTH)],
        [pltpu.VMEM((CHUNK,), jnp.float32) for _ in range(DEPTH)],
        [pltpu.VMEM((CHUNK,), jnp.float32) for _ in range(DEPTH)],
        [pltpu.SemaphoreType.DMA for _ in range(DEPTH)],
        [pltpu.SemaphoreType.DMA for _ in range(DEPTH)],
        [pltpu.SemaphoreType.DMA for _ in range(DEPTH)],
    ],
)
def _mod_add_sc(a_hbm, b_hbm, out_hbm, a_bufs, b_bufs, o_bufs,
                ina_sems, inb_sems, out_sems):
    wid = lax.axis_index("s") * NC + lax.axis_index("c")
    base = wid * PER_W

    def start_in(c, s):
        off = base + c * CHUNK
        pltpu.make_async_copy(
            a_hbm.at[pl.ds(off, CHUNK)], a_bufs[s], ina_sems[s]).start()
        pltpu.make_async_copy(
            b_hbm.at[pl.ds(off, CHUNK)], b_bufs[s], inb_sems[s]).start()

    def wait_in(s):
        pltpu.make_async_copy(
            a_hbm.at[pl.ds(base, CHUNK)], a_bufs[s], ina_sems[s]).wait()
        pltpu.make_async_copy(
            b_hbm.at[pl.ds(base, CHUNK)], b_bufs[s], inb_sems[s]).wait()

    def start_out(c, s):
        off = base + c * CHUNK
        pltpu.make_async_copy(
            o_bufs[s], out_hbm.at[pl.ds(off, CHUNK)], out_sems[s]).start()

    def wait_out(s):
        pltpu.make_async_copy(
            o_bufs[s], out_hbm.at[pl.ds(base, CHUNK)], out_sems[s]).wait()

    for s in range(DEPTH - 1):
        start_in(s, s)

    def ring_body(p, carry):
        for s in range(DEPTH):
            c = p * DEPTH + s

            @pl.when(c + DEPTH - 1 < NCHUNK)
            def _():
                start_in(c + DEPTH - 1, (s + DEPTH - 1) % DEPTH)

            wait_in(s)

            @pl.when(c >= DEPTH)
            def _():
                wait_out(s)

            a_buf, b_buf, o_buf = a_bufs[s], b_bufs[s], o_bufs[s]

            @plsc.parallel_loop(0, CHUNK, step=L, unroll=8)
            def _(j):
                av = a_buf[pl.ds(j, L)]
                bv = b_buf[pl.ds(j, L)]
                v = av + bv
                v = jnp.where(v >= 256.0, v - 256.0, v)
                v = jnp.where(v >= 256.0, v - 256.0, v)
                o_buf[pl.ds(j, L)] = v

            start_out(c, s)
        return carry

    lax.fori_loop(0, NCHUNK // DEPTH, ring_body, 0)
    for s in range(DEPTH):
        wait_out(s)


COLS = 1024
ROWS = N // COLS
HROWS = HALF // COLS
BR = 512


def _tc_body(a_ref, b_ref, o_ref):
    v = a_ref[...] + b_ref[...]
    v = jnp.where(v >= 256.0, v - 256.0, v)
    v = jnp.where(v >= 256.0, v - 256.0, v)
    o_ref[...] = v


_tc_half = pl.pallas_call(
    _tc_body,
    out_shape=jax.ShapeDtypeStruct((HROWS, COLS), jnp.float32),
    grid=(HROWS // BR,),
    in_specs=[
        pl.BlockSpec((BR, COLS), lambda i: (HROWS // BR + i, 0)),
        pl.BlockSpec((BR, COLS), lambda i: (HROWS // BR + i, 0)),
    ],
    out_specs=pl.BlockSpec((BR, COLS), lambda i: (i, 0)),
)


def kernel(a, b):
    sc_out = _mod_add_sc(a, b)
    tc_out = _tc_half(a.reshape(ROWS, COLS), b.reshape(ROWS, COLS))
    return jnp.concatenate([sc_out, tc_out.reshape(-1)])[:, None]


# speedup vs baseline: 3.2332x; 3.2332x over previous
import functools

import jax
import jax.numpy as jnp
from jax import lax
from jax.experimental import pallas as pl
from jax.experimental.pallas import tpu as pltpu, tpu_sc as plsc

N = 16777216
_INFO = plsc.get_sparse_core_info()
NC = _INFO.num_cores
NS = _INFO.num_subcores
L = _INFO.num_lanes
NW = NC * NS
PER_W = N // NW
CHUNK = 16368
NCHUNK = 32
TAIL = PER_W - NCHUNK * CHUNK
MAIN = CHUNK - CHUNK % (8 * L)
DEPTH = 4

_mesh = plsc.VectorSubcoreMesh(core_axis_name="c", subcore_axis_name="s")


@functools.partial(
    pl.kernel,
    out_type=jax.ShapeDtypeStruct((N,), jnp.float32),
    mesh=_mesh,
    scratch_types=[
        [pltpu.VMEM((CHUNK,), jnp.float32) for _ in range(DEPTH)],
        [pltpu.VMEM((CHUNK,), jnp.float32) for _ in range(DEPTH)],
        [pltpu.SemaphoreType.DMA for _ in range(DEPTH)],
        [pltpu.SemaphoreType.DMA for _ in range(DEPTH)],
        [pltpu.SemaphoreType.DMA for _ in range(DEPTH)],
    ],
)
def _mod_add_sc(a_hbm, b_hbm, out_hbm, a_bufs, b_bufs,
                ina_sems, inb_sems, out_sems):
    wid = lax.axis_index("s") * NC + lax.axis_index("c")
    base = wid * PER_W

    def start_in(c, s):
        off = base + c * CHUNK
        pltpu.make_async_copy(
            a_hbm.at[pl.ds(off, CHUNK)], a_bufs[s], ina_sems[s]).start()
        pltpu.make_async_copy(
            b_hbm.at[pl.ds(off, CHUNK)], b_bufs[s], inb_sems[s]).start()

    def wait_in(s):
        pltpu.make_async_copy(
            a_hbm.at[pl.ds(base, CHUNK)], a_bufs[s], ina_sems[s]).wait()
        pltpu.make_async_copy(
            b_hbm.at[pl.ds(base, CHUNK)], b_bufs[s], inb_sems[s]).wait()

    def start_out(c, s):
        off = base + c * CHUNK
        pltpu.make_async_copy(
            a_bufs[s], out_hbm.at[pl.ds(off, CHUNK)], out_sems[s]).start()

    def wait_out(s):
        pltpu.make_async_copy(
            a_bufs[s], out_hbm.at[pl.ds(base, CHUNK)], out_sems[s]).wait()

    for s in range(DEPTH - 1):
        start_in(s, s)

    def ring_body(p, carry):
        for s in range(DEPTH):
            c = p * DEPTH + s
            ns = (s + DEPTH - 1) % DEPTH

            @pl.when((c + DEPTH - 1 < NCHUNK) & (c >= 1))
            def _():
                wait_out(ns)

            @pl.when(c + DEPTH - 1 < NCHUNK)
            def _():
                start_in(c + DEPTH - 1, ns)

            wait_in(s)

            a_buf, b_buf = a_bufs[s], b_bufs[s]

            @plsc.parallel_loop(0, MAIN, step=L, unroll=8)
            def _(j):
                av = a_buf[pl.ds(j, L)]
                bv = b_buf[pl.ds(j, L)]
                v = av + bv
                v = jnp.where(v >= 256.0, v - 256.0, v)
                v = jnp.where(v >= 256.0, v - 256.0, v)
                a_buf[pl.ds(j, L)] = v

            @plsc.parallel_loop(MAIN, CHUNK, step=L, unroll=1)
            def _(j):
                av = a_buf[pl.ds(j, L)]
                bv = b_buf[pl.ds(j, L)]
                v = av + bv
                v = jnp.where(v >= 256.0, v - 256.0, v)
                v = jnp.where(v >= 256.0, v - 256.0, v)
                a_buf[pl.ds(j, L)] = v

            start_out(c, s)
        return carry

    lax.fori_loop(0, NCHUNK // DEPTH, ring_body, 0)
    for s in range(DEPTH):
        wait_out(s)

    toff = base + NCHUNK * CHUNK
    pltpu.sync_copy(a_hbm.at[pl.ds(toff, TAIL)], a_bufs[0].at[pl.ds(0, TAIL)])
    pltpu.sync_copy(b_hbm.at[pl.ds(toff, TAIL)], b_bufs[0].at[pl.ds(0, TAIL)])
    a_buf0, b_buf0 = a_bufs[0], b_bufs[0]

    @plsc.parallel_loop(0, TAIL, step=L, unroll=4)
    def _(j):
        av = a_buf0[pl.ds(j, L)]
        bv = b_buf0[pl.ds(j, L)]
        v = av + bv
        v = jnp.where(v >= 256.0, v - 256.0, v)
        v = jnp.where(v >= 256.0, v - 256.0, v)
        a_buf0[pl.ds(j, L)] = v

    pltpu.sync_copy(a_bufs[0].at[pl.ds(0, TAIL)], out_hbm.at[pl.ds(toff, TAIL)])


def kernel(a, b):
    out = _mod_add_sc(a, b)
    return out[:, None]


# speedup vs baseline: 3.2998x; 1.0206x over previous
import functools

import jax
import jax.numpy as jnp
from jax import lax
from jax.experimental import pallas as pl
from jax.experimental.pallas import tpu as pltpu, tpu_sc as plsc

N = 16777216
_INFO = plsc.get_sparse_core_info()
NC = _INFO.num_cores
NS = _INFO.num_subcores
L = _INFO.num_lanes
NW = NC * NS
PER_W = N // NW
CHUNK = 8192
NCHUNK = PER_W // CHUNK
DEPTH = 4

_mesh = plsc.VectorSubcoreMesh(core_axis_name="c", subcore_axis_name="s")


@functools.partial(
    pl.kernel,
    out_type=jax.ShapeDtypeStruct((N,), jnp.float32),
    mesh=_mesh,
    scratch_types=[
        [pltpu.VMEM((CHUNK,), jnp.float32) for _ in range(DEPTH)],
        [pltpu.VMEM((CHUNK,), jnp.float32) for _ in range(DEPTH)],
        [pltpu.VMEM((CHUNK,), jnp.float32) for _ in range(DEPTH)],
        [pltpu.SemaphoreType.DMA for _ in range(DEPTH)],
        [pltpu.SemaphoreType.DMA for _ in range(DEPTH)],
        [pltpu.SemaphoreType.DMA for _ in range(DEPTH)],
    ],
)
def _mod_add_sc(a_hbm, b_hbm, out_hbm, a_bufs, b_bufs, o_bufs,
                ina_sems, inb_sems, out_sems):
    wid = lax.axis_index("s") * NC + lax.axis_index("c")
    base = wid * PER_W

    def start_in(c, s):
        off = base + c * CHUNK
        pltpu.make_async_copy(
            a_hbm.at[pl.ds(off, CHUNK)], a_bufs[s], ina_sems[s]).start()
        pltpu.make_async_copy(
            b_hbm.at[pl.ds(off, CHUNK)], b_bufs[s], inb_sems[s]).start()

    def wait_in(s):
        pltpu.make_async_copy(
            a_hbm.at[pl.ds(base, CHUNK)], a_bufs[s], ina_sems[s]).wait()
        pltpu.make_async_copy(
            b_hbm.at[pl.ds(base, CHUNK)], b_bufs[s], inb_sems[s]).wait()

    def start_out(c, s):
        off = base + c * CHUNK
        pltpu.make_async_copy(
            o_bufs[s], out_hbm.at[pl.ds(off, CHUNK)], out_sems[s]).start()

    def wait_out(s):
        pltpu.make_async_copy(
            o_bufs[s], out_hbm.at[pl.ds(base, CHUNK)], out_sems[s]).wait()

    for s in range(DEPTH - 1):
        start_in(s, s)

    def ring_body(p, carry):
        for s in range(DEPTH):
            c = p * DEPTH + s

            @pl.when(c + DEPTH - 1 < NCHUNK)
            def _():
                start_in(c + DEPTH - 1, (s + DEPTH - 1) % DEPTH)

            wait_in(s)

            @pl.when(c >= DEPTH)
            def _():
                wait_out(s)

            a_buf, b_buf, o_buf = a_bufs[s], b_bufs[s], o_bufs[s]

            @plsc.parallel_loop(0, CHUNK, step=L, unroll=8)
            def _(j):
                av = a_buf[pl.ds(j, L)]
                bv = b_buf[pl.ds(j, L)]
                v = av + bv
                v = jnp.where(v >= 256.0, v - 256.0, v)
                v = jnp.where(v >= 256.0, v - 256.0, v)
                o_buf[pl.ds(j, L)] = v

            start_out(c, s)
        return carry

    lax.fori_loop(0, NCHUNK // DEPTH, ring_body, 0)
    for s in range(DEPTH):
        wait_out(s)


def kernel(a, b):
    out = _mod_add_sc(a, b)
    return out[:, None]


# speedup vs baseline: 3.3914x; 1.0278x over previous
import functools

import jax
import jax.numpy as jnp
from jax import lax
from jax.experimental import pallas as pl
from jax.experimental.pallas import tpu as pltpu, tpu_sc as plsc

N = 16777216
_INFO = plsc.get_sparse_core_info()
NC = _INFO.num_cores
NS = _INFO.num_subcores
L = _INFO.num_lanes
NW = NC * NS
PER_W = N // NW
CHUNK = 8192
NCHUNK = PER_W // CHUNK
DEPTH = 4

_mesh = plsc.VectorSubcoreMesh(core_axis_name="c", subcore_axis_name="s")


@functools.partial(
    pl.kernel,
    out_type=jax.ShapeDtypeStruct((N,), jnp.float32),
    mesh=_mesh,
    scratch_types=[
        [pltpu.VMEM((CHUNK,), jnp.float32) for _ in range(DEPTH)],
        [pltpu.VMEM((CHUNK,), jnp.float32) for _ in range(DEPTH)],
        [pltpu.VMEM((CHUNK,), jnp.float32) for _ in range(DEPTH)],
        pltpu.VMEM_SHARED((NS, DEPTH, CHUNK), jnp.float32),
        [pltpu.SemaphoreType.DMA for _ in range(DEPTH)],
        [pltpu.SemaphoreType.DMA for _ in range(DEPTH)],
        [pltpu.SemaphoreType.DMA for _ in range(DEPTH)],
        [pltpu.SemaphoreType.DMA for _ in range(DEPTH)],
    ],
)
def _mod_add_sc(a_hbm, b_hbm, out_hbm, a_bufs, b_bufs, o_bufs, sp_buf,
                ina_sems, inb_sems, s2s_sems, s2h_sems):
    sid = lax.axis_index("s")
    wid = sid * NC + lax.axis_index("c")
    base = wid * PER_W

    def start_in(c, s):
        off = base + c * CHUNK
        pltpu.make_async_copy(
            a_hbm.at[pl.ds(off, CHUNK)], a_bufs[s], ina_sems[s]).start()
        pltpu.make_async_copy(
            b_hbm.at[pl.ds(off, CHUNK)], b_bufs[s], inb_sems[s]).start()

    def wait_in(s):
        pltpu.make_async_copy(
            a_hbm.at[pl.ds(base, CHUNK)], a_bufs[s], ina_sems[s]).wait()
        pltpu.make_async_copy(
            b_hbm.at[pl.ds(base, CHUNK)], b_bufs[s], inb_sems[s]).wait()

    def start_s2s(s):
        pltpu.make_async_copy(
            o_bufs[s], sp_buf.at[sid, s], s2s_sems[s]).start()

    def wait_s2s(s):
        pltpu.make_async_copy(
            o_bufs[s], sp_buf.at[sid, s], s2s_sems[s]).wait()

    def start_s2h(c, s):
        off = base + c * CHUNK
        pltpu.make_async_copy(
            sp_buf.at[sid, s], out_hbm.at[pl.ds(off, CHUNK)],
            s2h_sems[s]).start()

    def wait_s2h(s):
        pltpu.make_async_copy(
            sp_buf.at[sid, s], out_hbm.at[pl.ds(base, CHUNK)],
            s2h_sems[s]).wait()

    for s in range(DEPTH - 1):
        start_in(s, s)

    def ring_body(p, carry):
        for s in range(DEPTH):
            c = p * DEPTH + s
            s1 = (s + DEPTH - 1) % DEPTH

            @pl.when(c + DEPTH - 1 < NCHUNK)
            def _():
                start_in(c + DEPTH - 1, s1)

            wait_in(s)

            @pl.when(c >= DEPTH)
            def _():
                wait_s2h(s)

            a_buf, b_buf, o_buf = a_bufs[s], b_bufs[s], o_bufs[s]

            @plsc.parallel_loop(0, CHUNK, step=L, unroll=8)
            def _(j):
                av = a_buf[pl.ds(j, L)]
                bv = b_buf[pl.ds(j, L)]
                v = av + bv
                v = jnp.where(v >= 256.0, v - 256.0, v)
                v = jnp.where(v >= 256.0, v - 256.0, v)
                o_buf[pl.ds(j, L)] = v

            start_s2s(s)

            @pl.when(c >= 1)
            def _():
                wait_s2s(s1)
                start_s2h(c - 1, s1)
        return carry

    lax.fori_loop(0, NCHUNK // DEPTH, ring_body, 0)
    last = NCHUNK - 1
    ls = last % DEPTH
    wait_s2s(ls)
    start_s2h(last, ls)
    for s in range(DEPTH):
        wait_s2h(s)


def kernel(a, b):
    out = _mod_add_sc(a, b)
    return out[:, None]
